# Initial kernel scaffold; baseline (speedup 1.0000x reference)
import jax
import jax.numpy as jnp
from jax.experimental import pallas as pl


def kernel(x, s, q, pos, senders, receivers, params):
    raise NotImplementedError("write your pallas kernel here")


# speedup vs baseline: 2.9148x; 2.9148x over previous
import functools

import jax
import jax.numpy as jnp
from jax import lax
from jax.experimental import pallas as pl
from jax.experimental.pallas import tpu as pltpu
from jax.experimental.pallas import tpu_sc as plsc

H = 128
_NC = 2
_NS = 16
_NW = _NC * _NS
_K = 80


def _full(shape):
    return pl.BlockSpec(shape, lambda i: (0,) * len(shape))


def _rows(bsize, ncols):
    return pl.BlockSpec((bsize, ncols), lambda i: (i, 0))


def _softmax_scores(pts, pos):
    cross = lax.dot_general(pts, pos, (((1,), (1,)), ((), ())))
    pn = jnp.sum(pos * pos, axis=1)[None, :]
    logits = 2.0 * cross - pn
    m = jnp.max(logits, axis=1, keepdims=True)
    e = jnp.exp(logits - m)
    return e / jnp.sum(e, axis=1, keepdims=True)


def _enc_latents_kernel(xs_ref, pos_ref, w0_ref, b0_ref, w1_ref, b1_ref,
                        w2_ref, b2_ref, out_ref):
    i = pl.program_id(0)
    xs = xs_ref[...]
    h = jnp.maximum(jnp.dot(xs, w0_ref[...]) + b0_ref[...], 0.0)
    h = jnp.maximum(jnp.dot(h, w1_ref[...]) + b1_ref[...], 0.0)
    emb = jnp.dot(h, w2_ref[...]) + b2_ref[...]
    scores = _softmax_scores(xs, pos_ref[...])
    contrib = lax.dot_general(scores, emb, (((0,), (0,)), ((), ())))

    @pl.when(i == 0)
    def _():
        out_ref[...] = jnp.zeros_like(out_ref)

    out_ref[...] += contrib


def _edge_enc_kernel(e0_ref, w0_ref, b0_ref, w1_ref, b1_ref,
                     w2_ref, b2_ref, out_ref):
    e0 = e0_ref[...]
    h = jnp.maximum(e0 * w0_ref[...] + b0_ref[...], 0.0)
    h = jnp.maximum(jnp.dot(h, w1_ref[...]) + b1_ref[...], 0.0)
    out_ref[...] = jnp.dot(h, w2_ref[...]) + b2_ref[...]


def _proj_kernel(n_ref, wr_ref, ws_ref, a_ref, b_ref):
    n = n_ref[...]
    a_ref[...] = jnp.dot(n, wr_ref[...])
    b_ref[...] = jnp.dot(n, ws_ref[...])


def _edge_mlp_kernel(e_ref, g1_ref, g2_ref, w0_ref, b0_ref, w1_ref, b1_ref,
                     w2_ref, b2_ref, out_ref):
    h = jnp.dot(e_ref[...], w0_ref[...]) + g1_ref[...] + g2_ref[...] + b0_ref[...]
    h = jnp.maximum(h, 0.0)
    h = jnp.maximum(jnp.dot(h, w1_ref[...]) + b1_ref[...], 0.0)
    out_ref[...] = jnp.dot(h, w2_ref[...]) + b2_ref[...]


def _node_kernel(n_ref, p0_ref, p1_ref, w0a_ref, w0b_ref, b0_ref, w1_ref,
                 b1_ref, w2_ref, b2_ref, out_ref):
    n = n_ref[...]
    inbox = p0_ref[...] + p1_ref[...]
    h = jnp.maximum(jnp.dot(n, w0a_ref[...]) + jnp.dot(inbox, w0b_ref[...])
                    + b0_ref[...], 0.0)
    h = jnp.maximum(jnp.dot(h, w1_ref[...]) + b1_ref[...], 0.0)
    out_ref[...] = n + jnp.dot(h, w2_ref[...]) + b2_ref[...]


def _decode_kernel(qs_ref, pos_ref, lat_ref, w0z_ref, w0q_ref, b0_ref,
                   w1_ref, b1_ref, w2_ref, b2_ref, out_ref):
    qs = qs_ref[...]
    scores = _softmax_scores(qs, pos_ref[...])
    z = jnp.dot(scores, lat_ref[...])
    h = jnp.maximum(jnp.dot(z, w0z_ref[...]) + jnp.dot(qs, w0q_ref[...])
                    + b0_ref[...], 0.0)
    h = jnp.maximum(jnp.dot(h, w1_ref[...]) + b1_ref[...], 0.0)
    out_ref[...] = jnp.dot(h, w2_ref[...]) + b2_ref[...]


def _sc_edge_len(E, NP):
    per_w = E // _NW
    n_chunks = per_w // _K
    sub = _K // 16
    mesh = plsc.VectorSubcoreMesh(core_axis_name="c", subcore_axis_name="s",
                                  num_cores=_NC, num_subcores=_NS)

    @functools.partial(
        pl.kernel,
        out_type=jax.ShapeDtypeStruct((E,), jnp.float32),
        mesh=mesh,
        compiler_params=pltpu.CompilerParams(needs_layout_passes=False),
        scratch_types=[
            pltpu.VMEM((NP,), jnp.float32),
            pltpu.VMEM((NP,), jnp.float32),
            pltpu.VMEM((NP,), jnp.float32),
            pltpu.VMEM((_K,), jnp.int32),
            pltpu.VMEM((_K,), jnp.int32),
            pltpu.VMEM((_K,), jnp.float32),
        ],
    )
    def elen(px_hbm, py_hbm, pz_hbm, ir_hbm, is_hbm, out_hbm,
             px_v, py_v, pz_v, ir_v, is_v, e_v):
        wid = lax.axis_index("s") * _NC + lax.axis_index("c")
        base = wid * per_w
        pltpu.sync_copy(px_hbm, px_v)
        pltpu.sync_copy(py_hbm, py_v)
        pltpu.sync_copy(pz_hbm, pz_v)

        def body(j, carry):
            off = base + j * _K
            pltpu.sync_copy(ir_hbm.at[pl.ds(off, _K)], ir_v)
            pltpu.sync_copy(is_hbm.at[pl.ds(off, _K)], is_v)

            def sub_body(t, carry2):
                ir = ir_v[pl.ds(t * 16, 16)]
                js = is_v[pl.ds(t * 16, 16)]
                dx = plsc.load_gather(px_v, [ir]) - plsc.load_gather(px_v, [js])
                dy = plsc.load_gather(py_v, [ir]) - plsc.load_gather(py_v, [js])
                dz = plsc.load_gather(pz_v, [ir]) - plsc.load_gather(pz_v, [js])
                e_v[pl.ds(t * 16, 16)] = dx * dx + dy * dy + dz * dz
                return carry2

            lax.fori_loop(0, sub, sub_body, 0)
            pltpu.sync_copy(e_v, out_hbm.at[pl.ds(off, _K)])
            return carry

        lax.fori_loop(0, n_chunks, body, 0)

    return elen


def _sc_gather2(E, D):
    per_w = E // _NW
    n_chunks = per_w // _K
    mesh = plsc.VectorSubcoreMesh(core_axis_name="c", subcore_axis_name="s",
                                  num_cores=_NC, num_subcores=_NS)

    @functools.partial(
        pl.kernel,
        out_type=(jax.ShapeDtypeStruct((E, D), jnp.float32),
                  jax.ShapeDtypeStruct((E, D), jnp.float32)),
        mesh=mesh,
        scratch_types=[
            pltpu.VMEM((_K,), jnp.int32),
            pltpu.VMEM((_K,), jnp.int32),
            pltpu.VMEM((_K, D), jnp.float32),
            pltpu.VMEM((_K, D), jnp.float32),
            pltpu.SemaphoreType.DMA,
            pltpu.SemaphoreType.DMA,
        ],
    )
    def gather2(t1_hbm, t2_hbm, i1_hbm, i2_hbm, o1_hbm, o2_hbm,
                i1_v, i2_v, r1_v, r2_v, sem1, sem2):
        wid = lax.axis_index("s") * _NC + lax.axis_index("c")
        base = wid * per_w

        def body(j, carry):
            off = base + j * _K
            pltpu.sync_copy(i1_hbm.at[pl.ds(off, _K)], i1_v)
            pltpu.sync_copy(i2_hbm.at[pl.ds(off, _K)], i2_v)
            c1 = pltpu.async_copy(t1_hbm.at[i1_v], r1_v, sem1)
            c2 = pltpu.async_copy(t2_hbm.at[i2_v], r2_v, sem2)
            c1.wait()
            c2.wait()
            pltpu.sync_copy(r1_v, o1_hbm.at[pl.ds(off, _K)])
            pltpu.sync_copy(r2_v, o2_hbm.at[pl.ds(off, _K)])
            return carry

        lax.fori_loop(0, n_chunks, body, 0)

    return gather2


def _sc_scatter_add(E, D, n_rows):
    per_w = E // _NW
    n_chunks = per_w // _K
    rows_per_tile = n_rows // _NS
    zchunks = rows_per_tile // _K
    mesh = plsc.VectorSubcoreMesh(core_axis_name="c", subcore_axis_name="s",
                                  num_cores=_NC, num_subcores=_NS)

    @functools.partial(
        pl.kernel,
        out_type=jax.ShapeDtypeStruct((_NC, n_rows, D), jnp.float32),
        mesh=mesh,
        scratch_types=[
            pltpu.VMEM((_K,), jnp.int32),
            pltpu.VMEM((_K, D), jnp.float32),
            pltpu.VMEM_SHARED((n_rows, D), jnp.float32),
        ],
    )
    def scatter(vals_hbm, idx_hbm, zeros_hbm, out_hbm, idx_v, rows_v, acc_sh):
        cid = lax.axis_index("c")
        sid = lax.axis_index("s")
        wid = sid * _NC + cid
        base = wid * per_w
        row0 = sid * rows_per_tile

        pltpu.sync_copy(zeros_hbm, rows_v)

        def zbody(j, carry):
            pltpu.sync_copy(rows_v, acc_sh.at[pl.ds(row0 + j * _K, _K)])
            return carry

        lax.fori_loop(0, zchunks, zbody, 0)
        plsc.subcore_barrier()

        def body(j, carry):
            off = base + j * _K
            pltpu.sync_copy(idx_hbm.at[pl.ds(off, _K)], idx_v)
            pltpu.sync_copy(vals_hbm.at[pl.ds(off, _K)], rows_v)
            pltpu.sync_copy(rows_v, acc_sh.at[idx_v], add=True)
            return carry

        lax.fori_loop(0, n_chunks, body, 0)
        plsc.subcore_barrier()

        def dbody(j, carry):
            r0 = row0 + j * _K
            pltpu.sync_copy(acc_sh.at[pl.ds(r0, _K)], rows_v)
            pltpu.sync_copy(rows_v, out_hbm.at[cid, pl.ds(r0, _K)])
            return carry

        lax.fori_loop(0, zchunks, dbody, 0)

    return scatter


def kernel(x, s, q, pos, senders, receivers, params):
    f32 = jnp.float32
    N = pos.shape[0]
    E = senders.shape[0]
    NX = x.shape[1]
    NP = ((N + _NS * _K - 1) // (_NS * _K)) * (_NS * _K)
    BX = 256
    BE = 2560
    BN = 2560

    senders = senders.astype(jnp.int32)
    receivers = receivers.astype(jnp.int32)

    pos8 = jnp.zeros((NP, 8), f32).at[:N, :3].set(pos).at[N:, 0].set(1e4)
    xs8 = jnp.zeros((NX, 8), f32).at[:, :3].set(x[0]).at[:, 3:6].set(s[0])
    q8 = jnp.zeros((NX, 8), f32).at[:, :3].set(q[0])
    posx = jnp.zeros((NP,), f32).at[:N].set(pos[:, 0])
    posy = jnp.zeros((NP,), f32).at[:N].set(pos[:, 1])
    posz = jnp.zeros((NP,), f32).at[:N].set(pos[:, 2])

    enc = params["encoder"]
    w0e = jnp.zeros((8, H), f32).at[:6].set(enc["W0"])
    latents = pl.pallas_call(
        _enc_latents_kernel,
        grid=(NX // BX,),
        in_specs=[_rows(BX, 8), _full((NP, 8)), _full((8, H)), _full((1, H)),
                  _full((H, H)), _full((1, H)), _full((H, H)), _full((1, H))],
        out_specs=_full((NP, H)),
        out_shape=jax.ShapeDtypeStruct((NP, H), f32),
    )(xs8, pos8, w0e, enc["b0"][None], enc["W1"], enc["b1"][None],
      enc["W2"], enc["b2"][None])

    e0 = _sc_edge_len(E, NP)(posx, posy, posz, receivers, senders)

    ee = params["edge_enc"]
    edges = pl.pallas_call(
        _edge_enc_kernel,
        grid=(E // BE,),
        in_specs=[_rows(BE, 1), _full((1, H)), _full((1, H)),
                  _full((H, H)), _full((1, H)), _full((H, H)), _full((1, H))],
        out_specs=_rows(BE, H),
        out_shape=jax.ShapeDtypeStruct((E, H), f32),
    )(e0[:, None], ee["W0"], ee["b0"][None], ee["W1"], ee["b1"][None],
      ee["W2"], ee["b2"][None])

    gather128 = _sc_gather2(E, H)
    scatter128 = _sc_scatter_add(E, H, NP)
    zeros_chunk = jnp.zeros((_K, H), f32)

    nodes = latents
    for bp in params["blocks"]:
        w0 = bp["edge"]["W0"]
        a, b = pl.pallas_call(
            _proj_kernel,
            grid=(NP // BN,),
            in_specs=[_rows(BN, H), _full((H, H)), _full((H, H))],
            out_specs=(_rows(BN, H), _rows(BN, H)),
            out_shape=(jax.ShapeDtypeStruct((NP, H), f32),
                       jax.ShapeDtypeStruct((NP, H), f32)),
        )(nodes, w0[H:2 * H], w0[2 * H:])

        g1, g2 = gather128(a, b, receivers, senders)

        eb = bp["edge"]
        edges = pl.pallas_call(
            _edge_mlp_kernel,
            grid=(E // BE,),
            in_specs=[_rows(BE, H), _rows(BE, H), _rows(BE, H), _full((H, H)),
                      _full((1, H)), _full((H, H)), _full((1, H)),
                      _full((H, H)), _full((1, H))],
            out_specs=_rows(BE, H),
            out_shape=jax.ShapeDtypeStruct((E, H), f32),
        )(edges, g1, g2, w0[:H], eb["b0"][None], eb["W1"], eb["b1"][None],
          eb["W2"], eb["b2"][None])

        parts = scatter128(edges, receivers, zeros_chunk)

        nd = bp["node"]
        nodes = pl.pallas_call(
            _node_kernel,
            grid=(NP // BN,),
            in_specs=[_rows(BN, H), _rows(BN, H), _rows(BN, H), _full((H, H)),
                      _full((H, H)), _full((1, H)), _full((H, H)),
                      _full((1, H)), _full((H, H)), _full((1, H))],
            out_specs=_rows(BN, H),
            out_shape=jax.ShapeDtypeStruct((NP, H), f32),
        )(nodes, parts[0], parts[1], nd["W0"][:H], nd["W0"][H:],
          nd["b0"][None], nd["W1"], nd["b1"][None], nd["W2"], nd["b2"][None])

    dec = params["decoder"]
    w0q = jnp.zeros((8, H), f32).at[:3].set(dec["W0"][H:H + 3])
    w2d = jnp.zeros((H, 8), f32).at[:, :3].set(dec["W2"])
    b2d = jnp.zeros((1, 8), f32).at[0, :3].set(dec["b2"])
    out8 = pl.pallas_call(
        _decode_kernel,
        grid=(NX // BX,),
        in_specs=[_rows(BX, 8), _full((NP, 8)), _full((NP, H)), _full((H, H)),
                  _full((8, H)), _full((1, H)), _full((H, H)), _full((1, H)),
                  _full((H, 8)), _full((1, 8))],
        out_specs=_rows(BX, 8),
        out_shape=jax.ShapeDtypeStruct((NX, 8), f32),
    )(q8, pos8, nodes, dec["W0"][:H], w0q, dec["b0"][None], dec["W1"],
      dec["b1"][None], w2d, b2d)

    return out8[:, :3].reshape(1, NX, 3)
